# baseline (speedup 1.0000x reference)
import functools

import jax
import jax.numpy as jnp
import numpy as np
from jax import lax
from jax.experimental import pallas as pl
from jax.experimental.pallas import tpu as pltpu
from jax.experimental.pallas import tpu_sc as plsc

D = 128
ROW = D + 1
GW = 16 * ROW


def _bf16_round(x):
    t = x * 65537.0
    return t - (t - x)

_GATHER_DNUMS = lax.GatherDimensionNumbers(
    offset_dims=(), collapsed_slice_dims=(0,), start_index_map=(0,))


def _shuffle(x, idx):
    return lax.gather(x, idx[:, None], _GATHER_DNUMS, (1,),
                      mode=lax.GatherScatterMode.PROMISE_IN_BOUNDS)


def _conc_kernel(l_ref, u_ref, lb_ref, ub_ref,
                 clb_ref, mlb_ref, mub_ref, cub_ref):
    lb = lb_ref[0]
    ub = ub_ref[0]

    lbr = _bf16_round(lb)[None, :]
    ubr = _bf16_round(ub)[None, :]

    def conc(x):
        w = x[:, :D]
        b = x[:, D]
        wr = _bf16_round(w)
        p1 = wr * lbr
        p2 = wr * ubr
        lo = jnp.sum(jnp.minimum(p1, p2), axis=1) + b
        hi = jnp.sum(jnp.maximum(p1, p2), axis=1) + b
        return lo, hi

    clb, mlb = conc(l_ref[...])
    mub, cub = conc(u_ref[...])
    bm = clb.shape[0]
    clb_ref[...] = clb.reshape(bm // 128, 128)
    mlb_ref[...] = mlb.reshape(bm // 128, 128)
    mub_ref[...] = mub.reshape(bm // 128, 128)
    cub_ref[...] = cub.reshape(bm // 128, 128)


def _concretize_tc(l2, u2, input_lb, input_ub, R, bm=2048):
    grid = (R // bm,)
    o = jax.ShapeDtypeStruct((R // 128, 128), jnp.float32)
    out = pl.pallas_call(
        _conc_kernel,
        grid=grid,
        in_specs=[
            pl.BlockSpec((bm, ROW), lambda i: (i, 0)),
            pl.BlockSpec((bm, ROW), lambda i: (i, 0)),
            pl.BlockSpec((1, D), lambda i: (0, 0)),
            pl.BlockSpec((1, D), lambda i: (0, 0)),
        ],
        out_specs=[pl.BlockSpec((bm // 128, 128), lambda i: (i, 0))] * 4,
        out_shape=[o, o, o, o],
    )(l2, u2, input_lb.reshape(1, D), input_ub.reshape(1, D))
    return out


def _make_sc_kernel(R, rows_per_worker, grp_per_chunk):
    chunk_rows = grp_per_chunk * 16
    chunk_w = grp_per_chunk * GW
    n_chunks = rows_per_worker // chunk_rows
    mesh = plsc.VectorSubcoreMesh(core_axis_name="c", subcore_axis_name="s")
    info = plsc.get_sparse_core_info()
    num_cores = info.num_cores

    @functools.partial(
        pl.kernel,
        mesh=mesh,
        out_type=[
            jax.ShapeDtypeStruct((R * ROW,), jnp.float32),
            jax.ShapeDtypeStruct((R * ROW,), jnp.float32),
            jax.ShapeDtypeStruct((R,), jnp.float32),
            jax.ShapeDtypeStruct((R,), jnp.float32),
        ],
        scratch_types=[
            pltpu.VMEM((chunk_w,), jnp.float32),
            pltpu.VMEM((chunk_w,), jnp.float32),
            pltpu.VMEM((rows_per_worker,), jnp.float32),
            pltpu.VMEM((rows_per_worker,), jnp.float32),
            pltpu.VMEM((rows_per_worker,), jnp.float32),
            pltpu.VMEM((rows_per_worker,), jnp.float32),
            pltpu.VMEM((rows_per_worker,), jnp.float32),
            pltpu.VMEM((rows_per_worker,), jnp.float32),
        ],
    )
    def sc_kernel(l_hbm, u_hbm, clb_hbm, mlb_hbm, mub_hbm, cub_hbm,
                  pl_hbm, pu_hbm, pclb_hbm, pcub_hbm,
                  il_v, iu_v,
                  clb_v, mlb_v, mub_v, cub_v, pclb_v, pcub_v):
        wid = lax.axis_index("s") * num_cores + lax.axis_index("c")
        row0 = wid * rows_per_worker
        w0 = row0 * ROW
        lane = lax.iota(jnp.int32, 16)
        pltpu.sync_copy(clb_hbm.at[pl.ds(row0, rows_per_worker)], clb_v)
        pltpu.sync_copy(mlb_hbm.at[pl.ds(row0, rows_per_worker)], mlb_v)
        pltpu.sync_copy(mub_hbm.at[pl.ds(row0, rows_per_worker)], mub_v)
        pltpu.sync_copy(cub_hbm.at[pl.ds(row0, rows_per_worker)], cub_v)

        def chunk_body(ci, _):
            cw = w0 + ci * chunk_w
            pltpu.sync_copy(l_hbm.at[pl.ds(cw, chunk_w)], il_v)
            pltpu.sync_copy(u_hbm.at[pl.ds(cw, chunk_w)], iu_v)

            def group_body(g, _):
                gr = ci * chunk_rows + g * 16
                conc_lb = clb_v[pl.ds(gr, 16)]
                max_lb = mlb_v[pl.ds(gr, 16)]
                min_ub = mub_v[pl.ds(gr, 16)]
                conc_ub = cub_v[pl.ds(gr, 16)]

                inactive = conc_ub <= 0.0
                unstable = (conc_lb < 0.0) & (conc_ub > 0.0)
                m_inact = unstable & (
                    (jnp.abs(conc_lb) > jnp.abs(conc_ub)) | (max_lb <= 0.0))
                m_act = unstable & (jnp.abs(conc_lb) <= jnp.abs(conc_ub))
                den_l = jnp.where(m_act, max_lb - conc_lb, 1.0)
                den_l = jnp.where(den_l == 0.0, 1.0, den_l)
                a_l = jnp.where(max_lb < 0.0, 0.0, max_lb / den_l)
                s_l = jnp.where(m_act, a_l,
                                jnp.where(inactive | m_inact, 0.0, 1.0))

                zc = unstable & (min_ub <= 0.0)
                den_u = jnp.where(zc, conc_ub - min_ub, 1.0)
                den_u = jnp.where(den_u == 0.0, 1.0, den_u)
                a_u = conc_ub / den_u
                s_u = jnp.where(zc, a_u, jnp.where(inactive, 0.0, 1.0))
                b_adj = jnp.where(zc, -a_u * min_ub, 0.0)

                pclb_v[pl.ds(gr, 16)] = jnp.maximum(s_l * conc_lb, 0.0)
                pcub_v[pl.ds(gr, 16)] = jnp.maximum(s_u * conc_ub + b_adj, 0.0)

                gw = g * GW
                splats = {}

                def spl(j):
                    if j not in splats:
                        splats[j] = (jnp.full((16,), s_l[j]),
                                     jnp.full((16,), s_u[j]))
                    return splats[j]

                for bb in range(ROW):
                    r1 = (16 * bb) // ROW
                    r2 = (16 * bb + 15) // ROW
                    sl1, su1 = spl(r1)
                    if r2 == r1:
                        slv, suv = sl1, su1
                    else:
                        cut = r2 * ROW - 16 * bb
                        m = lane < cut
                        sl2, su2 = spl(r2)
                        slv = jnp.where(m, sl1, sl2)
                        suv = jnp.where(m, su1, su2)
                    sl = pl.ds(gw + bb * 16, 16)
                    il_v[sl] = slv * il_v[sl]
                    ub_blk = suv * iu_v[sl]
                    for j in sorted(set((r1, r2))):
                        bpos = j * ROW + D
                        if 16 * bb <= bpos < 16 * bb + 16:
                            ub_blk = ub_blk + jnp.where(
                                lane == (bpos - 16 * bb),
                                jnp.full((16,), b_adj[j]), 0.0)
                    iu_v[sl] = ub_blk
                return 0

            lax.fori_loop(0, grp_per_chunk, group_body, 0)
            pltpu.sync_copy(il_v, pl_hbm.at[pl.ds(cw, chunk_w)])
            pltpu.sync_copy(iu_v, pu_hbm.at[pl.ds(cw, chunk_w)])
            return 0

        lax.fori_loop(0, n_chunks, chunk_body, 0)
        pltpu.sync_copy(pclb_v, pclb_hbm.at[pl.ds(row0, rows_per_worker)])
        pltpu.sync_copy(pcub_v, pcub_hbm.at[pl.ds(row0, rows_per_worker)])

    return sc_kernel


def kernel(l, u, input_lb, input_ub):
    B, N, row = l.shape
    R = B * N
    n_workers = 32
    rows_per_worker = R // n_workers
    l2 = l.reshape(R, row)
    u2 = u.reshape(R, row)
    clb, mlb, mub, cub = _concretize_tc(l2, u2, input_lb, input_ub, R)
    sc = _make_sc_kernel(R, rows_per_worker, grp_per_chunk=16)
    post_l, post_u, pclb, pcub = sc(
        l2.reshape(R * row), u2.reshape(R * row),
        clb.reshape(R), mlb.reshape(R), mub.reshape(R), cub.reshape(R))
    return (post_l.reshape(B, N, row), post_u.reshape(B, N, row),
            pclb.reshape(B, N), pcub.reshape(B, N))


# speedup vs baseline: 1.5951x; 1.5951x over previous
import functools

import jax
import jax.numpy as jnp
import numpy as np
from jax import lax
from jax.experimental import pallas as pl
from jax.experimental.pallas import tpu as pltpu
from jax.experimental.pallas import tpu_sc as plsc

D = 128
ROW = D + 1
GW = 16 * ROW


def _bf16_round(x):
    t = x * 65537.0
    return t - (t - x)

_GATHER_DNUMS = lax.GatherDimensionNumbers(
    offset_dims=(), collapsed_slice_dims=(0,), start_index_map=(0,))


def _shuffle(x, idx):
    return lax.gather(x, idx[:, None], _GATHER_DNUMS, (1,),
                      mode=lax.GatherScatterMode.PROMISE_IN_BOUNDS)


def _conc_kernel(l_ref, u_ref, lb_ref, ub_ref,
                 clb_ref, mlb_ref, mub_ref, cub_ref):
    lb = lb_ref[0]
    ub = ub_ref[0]

    lbr = _bf16_round(lb)[None, :]
    ubr = _bf16_round(ub)[None, :]

    def conc(x):
        w = x[:, :D]
        b = x[:, D]
        wr = _bf16_round(w)
        p1 = wr * lbr
        p2 = wr * ubr
        lo = jnp.sum(jnp.minimum(p1, p2), axis=1) + b
        hi = jnp.sum(jnp.maximum(p1, p2), axis=1) + b
        return lo, hi

    clb, mlb = conc(l_ref[...])
    mub, cub = conc(u_ref[...])
    bm = clb.shape[0]
    clb_ref[...] = clb.reshape(bm // 128, 128)
    mlb_ref[...] = mlb.reshape(bm // 128, 128)
    mub_ref[...] = mub.reshape(bm // 128, 128)
    cub_ref[...] = cub.reshape(bm // 128, 128)


def _concretize_tc(l2, u2, input_lb, input_ub, R, bm=2048):
    grid = (R // bm,)
    o = jax.ShapeDtypeStruct((R // 128, 128), jnp.float32)
    out = pl.pallas_call(
        _conc_kernel,
        grid=grid,
        in_specs=[
            pl.BlockSpec((bm, ROW), lambda i: (i, 0)),
            pl.BlockSpec((bm, ROW), lambda i: (i, 0)),
            pl.BlockSpec((1, D), lambda i: (0, 0)),
            pl.BlockSpec((1, D), lambda i: (0, 0)),
        ],
        out_specs=[pl.BlockSpec((bm // 128, 128), lambda i: (i, 0))] * 4,
        out_shape=[o, o, o, o],
    )(l2, u2, input_lb.reshape(1, D), input_ub.reshape(1, D))
    return out


def _make_sc_kernel(R, rows_per_worker, grp_per_chunk):
    chunk_rows = grp_per_chunk * 16
    chunk_w = grp_per_chunk * GW
    n_chunks = rows_per_worker // chunk_rows
    mesh = plsc.VectorSubcoreMesh(core_axis_name="c", subcore_axis_name="s")
    info = plsc.get_sparse_core_info()
    num_cores = info.num_cores

    @functools.partial(
        pl.kernel,
        mesh=mesh,
        out_type=[
            jax.ShapeDtypeStruct((R,), jnp.float32),
            jax.ShapeDtypeStruct((R,), jnp.float32),
            jax.ShapeDtypeStruct((R,), jnp.float32),
            jax.ShapeDtypeStruct((R,), jnp.float32),
            jax.ShapeDtypeStruct((R,), jnp.float32),
        ],
        scratch_types=[
            pltpu.VMEM((rows_per_worker,), jnp.float32),
            pltpu.VMEM((rows_per_worker,), jnp.float32),
            pltpu.VMEM((rows_per_worker,), jnp.float32),
            pltpu.VMEM((rows_per_worker,), jnp.float32),
            pltpu.VMEM((rows_per_worker,), jnp.float32),
            pltpu.VMEM((rows_per_worker,), jnp.float32),
            pltpu.VMEM((rows_per_worker,), jnp.float32),
            pltpu.VMEM((rows_per_worker,), jnp.float32),
            pltpu.VMEM((rows_per_worker,), jnp.float32),
        ],
    )
    def sc_kernel(clb_hbm, mlb_hbm, mub_hbm, cub_hbm,
                  sl_hbm, su_hbm, ba_hbm, pclb_hbm, pcub_hbm,
                  clb_v, mlb_v, mub_v, cub_v,
                  sl_v, su_v, ba_v, pclb_v, pcub_v):
        wid = lax.axis_index("s") * num_cores + lax.axis_index("c")
        row0 = wid * rows_per_worker
        pltpu.sync_copy(clb_hbm.at[pl.ds(row0, rows_per_worker)], clb_v)
        pltpu.sync_copy(mlb_hbm.at[pl.ds(row0, rows_per_worker)], mlb_v)
        pltpu.sync_copy(mub_hbm.at[pl.ds(row0, rows_per_worker)], mub_v)
        pltpu.sync_copy(cub_hbm.at[pl.ds(row0, rows_per_worker)], cub_v)

        if True:

            def group_body(g, _):
                gr = g * 16
                conc_lb = clb_v[pl.ds(gr, 16)]
                max_lb = mlb_v[pl.ds(gr, 16)]
                min_ub = mub_v[pl.ds(gr, 16)]
                conc_ub = cub_v[pl.ds(gr, 16)]

                inactive = conc_ub <= 0.0
                unstable = (conc_lb < 0.0) & (conc_ub > 0.0)
                m_inact = unstable & (
                    (jnp.abs(conc_lb) > jnp.abs(conc_ub)) | (max_lb <= 0.0))
                m_act = unstable & (jnp.abs(conc_lb) <= jnp.abs(conc_ub))
                den_l = jnp.where(m_act, max_lb - conc_lb, 1.0)
                den_l = jnp.where(den_l == 0.0, 1.0, den_l)
                a_l = jnp.where(max_lb < 0.0, 0.0, max_lb / den_l)
                s_l = jnp.where(m_act, a_l,
                                jnp.where(inactive | m_inact, 0.0, 1.0))

                zc = unstable & (min_ub <= 0.0)
                den_u = jnp.where(zc, conc_ub - min_ub, 1.0)
                den_u = jnp.where(den_u == 0.0, 1.0, den_u)
                a_u = conc_ub / den_u
                s_u = jnp.where(zc, a_u, jnp.where(inactive, 0.0, 1.0))
                b_adj = jnp.where(zc, -a_u * min_ub, 0.0)

                sl_v[pl.ds(gr, 16)] = s_l
                su_v[pl.ds(gr, 16)] = s_u
                ba_v[pl.ds(gr, 16)] = b_adj
                pclb_v[pl.ds(gr, 16)] = jnp.maximum(s_l * conc_lb, 0.0)
                pcub_v[pl.ds(gr, 16)] = jnp.maximum(s_u * conc_ub + b_adj, 0.0)
                return 0

            lax.fori_loop(0, rows_per_worker // 16, group_body, 0)

        pltpu.sync_copy(sl_v, sl_hbm.at[pl.ds(row0, rows_per_worker)])
        pltpu.sync_copy(su_v, su_hbm.at[pl.ds(row0, rows_per_worker)])
        pltpu.sync_copy(ba_v, ba_hbm.at[pl.ds(row0, rows_per_worker)])
        pltpu.sync_copy(pclb_v, pclb_hbm.at[pl.ds(row0, rows_per_worker)])
        pltpu.sync_copy(pcub_v, pcub_hbm.at[pl.ds(row0, rows_per_worker)])

    return sc_kernel


def _scale_kernel(l_ref, u_ref, sl_ref, su_ref, ba_ref, ol_ref, ou_ref):
    slT = jnp.transpose(sl_ref[...])
    suT = jnp.transpose(su_ref[...])
    baT = jnp.transpose(ba_ref[...])
    col = lax.broadcasted_iota(jnp.int32, (1, ROW), 1)
    bias_col = col == D
    nb = slT.shape[1]
    for j in range(nb):
        band = pl.ds(j * 128, 128)
        s_l = slT[:, j:j + 1]
        s_u = suT[:, j:j + 1]
        ba = baT[:, j:j + 1]
        ol_ref[band, :] = s_l * l_ref[band, :]
        ou_ref[band, :] = (s_u * u_ref[band, :]
                           + jnp.where(bias_col, ba, 0.0))


def _scale_tc(l2, u2, s_l, s_u, badj, R, bm=2048):
    grid = (R // bm,)
    o = jax.ShapeDtypeStruct((R, ROW), jnp.float32)
    return pl.pallas_call(
        _scale_kernel,
        grid=grid,
        in_specs=[
            pl.BlockSpec((bm, ROW), lambda i: (i, 0)),
            pl.BlockSpec((bm, ROW), lambda i: (i, 0)),
            pl.BlockSpec((bm // 128, 128), lambda i: (i, 0)),
            pl.BlockSpec((bm // 128, 128), lambda i: (i, 0)),
            pl.BlockSpec((bm // 128, 128), lambda i: (i, 0)),
        ],
        out_specs=[pl.BlockSpec((bm, ROW), lambda i: (i, 0))] * 2,
        out_shape=[o, o],
    )(l2, u2, s_l, s_u, badj)


def kernel(l, u, input_lb, input_ub):
    B, N, row = l.shape
    R = B * N
    n_workers = 32
    rows_per_worker = R // n_workers
    l2 = l.reshape(R, row)
    u2 = u.reshape(R, row)
    clb, mlb, mub, cub = _concretize_tc(l2, u2, input_lb, input_ub, R)
    sc = _make_sc_kernel(R, rows_per_worker, grp_per_chunk=16)
    s_l, s_u, badj, pclb, pcub = sc(
        clb.reshape(R), mlb.reshape(R), mub.reshape(R), cub.reshape(R))
    post_l, post_u = _scale_tc(
        l2, u2, s_l.reshape(R // 128, 128), s_u.reshape(R // 128, 128),
        badj.reshape(R // 128, 128), R)
    return (post_l.reshape(B, N, row), post_u.reshape(B, N, row),
            pclb.reshape(B, N), pcub.reshape(B, N))


# speedup vs baseline: 1.6066x; 1.0072x over previous
import functools

import jax
import jax.numpy as jnp
import numpy as np
from jax import lax
from jax.experimental import pallas as pl
from jax.experimental.pallas import tpu as pltpu
from jax.experimental.pallas import tpu_sc as plsc

D = 128
ROW = D + 1
GW = 16 * ROW


def _bf16_round(x):
    t = x * 65537.0
    return t - (t - x)

_GATHER_DNUMS = lax.GatherDimensionNumbers(
    offset_dims=(), collapsed_slice_dims=(0,), start_index_map=(0,))


def _shuffle(x, idx):
    return lax.gather(x, idx[:, None], _GATHER_DNUMS, (1,),
                      mode=lax.GatherScatterMode.PROMISE_IN_BOUNDS)


def _conc_kernel(l_ref, u_ref, lb_ref, ub_ref,
                 clb_ref, mlb_ref, mub_ref, cub_ref):
    lb = lb_ref[0]
    ub = ub_ref[0]

    lbr = _bf16_round(lb)[None, :]
    ubr = _bf16_round(ub)[None, :]

    def conc(x):
        w = x[:, :D]
        b = x[:, D]
        wr = _bf16_round(w)
        p1 = wr * lbr
        p2 = wr * ubr
        lo = jnp.sum(jnp.minimum(p1, p2), axis=1) + b
        hi = jnp.sum(jnp.maximum(p1, p2), axis=1) + b
        return lo, hi

    clb, mlb = conc(l_ref[...])
    mub, cub = conc(u_ref[...])
    bm = clb.shape[0]
    clb_ref[...] = clb.reshape(bm // 128, 128)
    mlb_ref[...] = mlb.reshape(bm // 128, 128)
    mub_ref[...] = mub.reshape(bm // 128, 128)
    cub_ref[...] = cub.reshape(bm // 128, 128)


def _concretize_tc(l2, u2, input_lb, input_ub, R, bm=4096):
    grid = (R // bm,)
    o = jax.ShapeDtypeStruct((R // 128, 128), jnp.float32)
    out = pl.pallas_call(
        _conc_kernel,
        grid=grid,
        in_specs=[
            pl.BlockSpec((bm, ROW), lambda i: (i, 0)),
            pl.BlockSpec((bm, ROW), lambda i: (i, 0)),
            pl.BlockSpec((1, D), lambda i: (0, 0)),
            pl.BlockSpec((1, D), lambda i: (0, 0)),
        ],
        out_specs=[pl.BlockSpec((bm // 128, 128), lambda i: (i, 0))] * 4,
        out_shape=[o, o, o, o],
    )(l2, u2, input_lb.reshape(1, D), input_ub.reshape(1, D))
    return out


def _make_sc_kernel(R, rows_per_worker, grp_per_chunk):
    chunk_rows = grp_per_chunk * 16
    chunk_w = grp_per_chunk * GW
    n_chunks = rows_per_worker // chunk_rows
    mesh = plsc.VectorSubcoreMesh(core_axis_name="c", subcore_axis_name="s")
    info = plsc.get_sparse_core_info()
    num_cores = info.num_cores

    @functools.partial(
        pl.kernel,
        mesh=mesh,
        out_type=[
            jax.ShapeDtypeStruct((R,), jnp.float32),
            jax.ShapeDtypeStruct((R,), jnp.float32),
            jax.ShapeDtypeStruct((R,), jnp.float32),
            jax.ShapeDtypeStruct((R,), jnp.float32),
            jax.ShapeDtypeStruct((R,), jnp.float32),
        ],
        scratch_types=[
            pltpu.VMEM((rows_per_worker,), jnp.float32),
            pltpu.VMEM((rows_per_worker,), jnp.float32),
            pltpu.VMEM((rows_per_worker,), jnp.float32),
            pltpu.VMEM((rows_per_worker,), jnp.float32),
            pltpu.VMEM((rows_per_worker,), jnp.float32),
            pltpu.VMEM((rows_per_worker,), jnp.float32),
            pltpu.VMEM((rows_per_worker,), jnp.float32),
            pltpu.VMEM((rows_per_worker,), jnp.float32),
            pltpu.VMEM((rows_per_worker,), jnp.float32),
        ],
    )
    def sc_kernel(clb_hbm, mlb_hbm, mub_hbm, cub_hbm,
                  sl_hbm, su_hbm, ba_hbm, pclb_hbm, pcub_hbm,
                  clb_v, mlb_v, mub_v, cub_v,
                  sl_v, su_v, ba_v, pclb_v, pcub_v):
        wid = lax.axis_index("s") * num_cores + lax.axis_index("c")
        row0 = wid * rows_per_worker
        pltpu.sync_copy(clb_hbm.at[pl.ds(row0, rows_per_worker)], clb_v)
        pltpu.sync_copy(mlb_hbm.at[pl.ds(row0, rows_per_worker)], mlb_v)
        pltpu.sync_copy(mub_hbm.at[pl.ds(row0, rows_per_worker)], mub_v)
        pltpu.sync_copy(cub_hbm.at[pl.ds(row0, rows_per_worker)], cub_v)

        if True:

            def group_body(g, _):
                gr = g * 16
                conc_lb = clb_v[pl.ds(gr, 16)]
                max_lb = mlb_v[pl.ds(gr, 16)]
                min_ub = mub_v[pl.ds(gr, 16)]
                conc_ub = cub_v[pl.ds(gr, 16)]

                inactive = conc_ub <= 0.0
                unstable = (conc_lb < 0.0) & (conc_ub > 0.0)
                m_inact = unstable & (
                    (jnp.abs(conc_lb) > jnp.abs(conc_ub)) | (max_lb <= 0.0))
                m_act = unstable & (jnp.abs(conc_lb) <= jnp.abs(conc_ub))
                den_l = jnp.where(m_act, max_lb - conc_lb, 1.0)
                den_l = jnp.where(den_l == 0.0, 1.0, den_l)
                a_l = jnp.where(max_lb < 0.0, 0.0, max_lb / den_l)
                s_l = jnp.where(m_act, a_l,
                                jnp.where(inactive | m_inact, 0.0, 1.0))

                zc = unstable & (min_ub <= 0.0)
                den_u = jnp.where(zc, conc_ub - min_ub, 1.0)
                den_u = jnp.where(den_u == 0.0, 1.0, den_u)
                a_u = conc_ub / den_u
                s_u = jnp.where(zc, a_u, jnp.where(inactive, 0.0, 1.0))
                b_adj = jnp.where(zc, -a_u * min_ub, 0.0)

                sl_v[pl.ds(gr, 16)] = s_l
                su_v[pl.ds(gr, 16)] = s_u
                ba_v[pl.ds(gr, 16)] = b_adj
                pclb_v[pl.ds(gr, 16)] = jnp.maximum(s_l * conc_lb, 0.0)
                pcub_v[pl.ds(gr, 16)] = jnp.maximum(s_u * conc_ub + b_adj, 0.0)
                return 0

            lax.fori_loop(0, rows_per_worker // 16, group_body, 0)

        pltpu.sync_copy(sl_v, sl_hbm.at[pl.ds(row0, rows_per_worker)])
        pltpu.sync_copy(su_v, su_hbm.at[pl.ds(row0, rows_per_worker)])
        pltpu.sync_copy(ba_v, ba_hbm.at[pl.ds(row0, rows_per_worker)])
        pltpu.sync_copy(pclb_v, pclb_hbm.at[pl.ds(row0, rows_per_worker)])
        pltpu.sync_copy(pcub_v, pcub_hbm.at[pl.ds(row0, rows_per_worker)])

    return sc_kernel


def _scale_kernel(l_ref, u_ref, sl_ref, su_ref, ba_ref, ol_ref, ou_ref):
    slT = jnp.transpose(sl_ref[...])
    suT = jnp.transpose(su_ref[...])
    baT = jnp.transpose(ba_ref[...])
    col = lax.broadcasted_iota(jnp.int32, (1, ROW), 1)
    bias_col = col == D
    nb = slT.shape[1]
    for j in range(nb):
        band = pl.ds(j * 128, 128)
        s_l = slT[:, j:j + 1]
        s_u = suT[:, j:j + 1]
        ba = baT[:, j:j + 1]
        ol_ref[band, :] = s_l * l_ref[band, :]
        ou_ref[band, :] = (s_u * u_ref[band, :]
                           + jnp.where(bias_col, ba, 0.0))


def _scale_tc(l2, u2, s_l, s_u, badj, R, bm=4096):
    grid = (R // bm,)
    o = jax.ShapeDtypeStruct((R, ROW), jnp.float32)
    return pl.pallas_call(
        _scale_kernel,
        grid=grid,
        in_specs=[
            pl.BlockSpec((bm, ROW), lambda i: (i, 0)),
            pl.BlockSpec((bm, ROW), lambda i: (i, 0)),
            pl.BlockSpec((bm // 128, 128), lambda i: (i, 0)),
            pl.BlockSpec((bm // 128, 128), lambda i: (i, 0)),
            pl.BlockSpec((bm // 128, 128), lambda i: (i, 0)),
        ],
        out_specs=[pl.BlockSpec((bm, ROW), lambda i: (i, 0))] * 2,
        out_shape=[o, o],
    )(l2, u2, s_l, s_u, badj)


def kernel(l, u, input_lb, input_ub):
    B, N, row = l.shape
    R = B * N
    n_workers = 32
    rows_per_worker = R // n_workers
    l2 = l.reshape(R, row)
    u2 = u.reshape(R, row)
    clb, mlb, mub, cub = _concretize_tc(l2, u2, input_lb, input_ub, R)
    sc = _make_sc_kernel(R, rows_per_worker, grp_per_chunk=16)
    s_l, s_u, badj, pclb, pcub = sc(
        clb.reshape(R), mlb.reshape(R), mub.reshape(R), cub.reshape(R))
    post_l, post_u = _scale_tc(
        l2, u2, s_l.reshape(R // 128, 128), s_u.reshape(R // 128, 128),
        badj.reshape(R // 128, 128), R)
    return (post_l.reshape(B, N, row), post_u.reshape(B, N, row),
            pclb.reshape(B, N), pcub.reshape(B, N))
